# baseline (speedup 1.0000x reference)
import functools

import jax
import jax.numpy as jnp
from jax import lax
from jax.experimental import pallas as pl
from jax.experimental.pallas import tpu as pltpu
from jax.experimental.pallas import tpu_sc as plsc


def kernel(x, embed_table):
    b, n, m = x.shape
    e = embed_table.shape[-1]
    nt = n // 8
    xt = jnp.transpose(x, (1, 2, 0))
    tab2 = jnp.reshape(embed_table, (n // 2, m + e))

    mesh = plsc.VectorSubcoreMesh(core_axis_name="c", subcore_axis_name="s")

    @functools.partial(
        pl.kernel,
        out_type=jax.ShapeDtypeStruct((b, n, m + e), jnp.float32),
        mesh=mesh,
        scratch_types=[
            *[pltpu.VMEM((8, m // 2, 128), jnp.float32) for _ in range(2)],
            *[pltpu.VMEM((16, 8, m + e), jnp.float32) for _ in range(2)],
            pltpu.VMEM((n // 2, m + e), jnp.float32),
            *[pltpu.SemaphoreType.DMA for _ in range(5)],
        ],
        compiler_params=pltpu.CompilerParams(
            use_tc_tiling_on_sc=True, needs_layout_passes=False),
    )
    def run(xt_hbm, tab_hbm, out_hbm, xb0, xb1, ob0, ob1, tab_v, *sems):
        xbs, obs = (xb0, xb1), (ob0, ob1)
        sx = sems[:2]
        so = sems[2:4]
        st = sems[4]

        wid = lax.axis_index("s") * 2 + lax.axis_index("c")
        t0 = wid // 8
        b0 = pl.multiple_of((wid % 8) * 128, 128)

        lane = lax.iota(jnp.int32, 16)

        def in_cp(slot, t, h):
            return pltpu.make_async_copy(
                xt_hbm.at[pl.ds(8 * t, 8), pl.ds(h * (m // 2), m // 2),
                          pl.ds(b0, 128)],
                xbs[slot], sx[slot])

        def out_cp(buf, t, bi, jj):
            return pltpu.make_async_copy(
                obs[buf].at[jj], out_hbm.at[bi, pl.ds(8 * t, 8), :], so[buf])

        in_cp(0, t0, 0).start()
        in_cp(1, t0, 1).start()
        pltpu.make_async_copy(tab_hbm, tab_v, st).start()
        pltpu.make_async_copy(tab_hbm, tab_v, st).wait()

        def item(k, carry):
            t = t0 + 4 * k

            @pl.when(t < nt)
            def _():
                def prefill(j, cp):
                    for r in range(8):
                        for c in range(0, m, 16):
                            v = tab_v[4 * t + r // 2,
                                      pl.ds((r % 2) * m + c, 16)]
                            obs[0][j, r, pl.ds(m + c, 16)] = v
                            obs[1][j, r, pl.ds(m + c, 16)] = v
                    return cp

                lax.fori_loop(0, 16, prefill, 0)
                in_cp(0, t, 0).wait()
                in_cp(1, t, 1).wait()

                def body(buf, g):
                    @pl.when(g >= 2)
                    def _():
                        for j in range(16):
                            out_cp(buf, t, b0 + (g - 2) * 16 + j, j).wait()
                    gbase = g * 16

                    def sweep(jj, cp):
                        for u in range(2):
                            perm_j = lax.rem(lane + (2 * jj + u), 16)
                            bidx = gbase + perm_j
                            for r2 in range(0, 8, 4):
                                vs = []
                                for r in range(r2, r2 + 4):
                                    for h in range(2):
                                        for c in range(0, m // 2, 16):
                                            cc = h * (m // 2) + c
                                            vs.append((r, cc, plsc.load_gather(
                                                xbs[h].at[r],
                                                [c + lane, bidx])))
                                for r, cc, v in vs:
                                    plsc.store_scatter(
                                        obs[buf],
                                        [perm_j,
                                         jnp.full((16,), r, jnp.int32),
                                         cc + lane], v)
                        return cp

                    lax.fori_loop(0, 8, sweep, 0)
                    for j in range(16):
                        out_cp(buf, t, b0 + gbase + j, j).start()

                def group(g, carry2):
                    @pl.when(lax.rem(g, 2) == 0)
                    def _():
                        body(0, g)

                    @pl.when(lax.rem(g, 2) == 1)
                    def _():
                        body(1, g)

                    return carry2

                lax.fori_loop(0, 8, group, 0)

                @pl.when(t + 4 < nt)
                def _():
                    in_cp(0, t + 4, 0).start()
                    in_cp(1, t + 4, 1).start()

                for j in range(16):
                    out_cp(0, t, b0 + 6 * 16 + j, j).wait()
                    out_cp(1, t, b0 + 7 * 16 + j, j).wait()

            return carry

        lax.fori_loop(0, 7, item, 0)

    return run(xt, tab2)


# speedup vs baseline: 1.0170x; 1.0170x over previous
import functools

import jax
import jax.numpy as jnp
from jax import lax
from jax.experimental import pallas as pl
from jax.experimental.pallas import tpu as pltpu
from jax.experimental.pallas import tpu_sc as plsc


def kernel(x, embed_table):
    b, n, m = x.shape
    e = embed_table.shape[-1]
    nt = n // 8
    xt = jnp.transpose(x, (1, 2, 0))
    tab2 = jnp.reshape(embed_table, (n // 2, m + e))

    mesh = plsc.VectorSubcoreMesh(core_axis_name="c", subcore_axis_name="s")

    @functools.partial(
        pl.kernel,
        out_type=jax.ShapeDtypeStruct((b, n, m + e), jnp.float32),
        mesh=mesh,
        scratch_types=[
            *[pltpu.VMEM((8, m // 2, 128), jnp.float32) for _ in range(2)],
            *[pltpu.VMEM((16, 8, m + e), jnp.float32) for _ in range(2)],
            pltpu.VMEM((n // 2, m + e), jnp.float32),
            *[pltpu.SemaphoreType.DMA for _ in range(5)],
        ],
        compiler_params=pltpu.CompilerParams(
            use_tc_tiling_on_sc=True, needs_layout_passes=False),
    )
    def run(xt_hbm, tab_hbm, out_hbm, xb0, xb1, ob0, ob1, tab_v, *sems):
        xbs, obs = (xb0, xb1), (ob0, ob1)
        sx = sems[:2]
        so = sems[2:4]
        st = sems[4]

        wid = lax.axis_index("s") * 2 + lax.axis_index("c")
        t0 = wid // 8
        b0 = pl.multiple_of((wid % 8) * 128, 128)

        lane = lax.iota(jnp.int32, 16)

        def in_cp(slot, t, h):
            return pltpu.make_async_copy(
                xt_hbm.at[pl.ds(8 * t, 8), pl.ds(h * (m // 2), m // 2),
                          pl.ds(b0, 128)],
                xbs[slot], sx[slot])

        def out_cp(buf, t, bi, jj):
            return pltpu.make_async_copy(
                obs[buf].at[jj], out_hbm.at[bi, pl.ds(8 * t, 8), :], so[buf])

        in_cp(0, t0, 0).start()
        in_cp(1, t0, 1).start()
        pltpu.make_async_copy(tab_hbm, tab_v, st).start()
        pltpu.make_async_copy(tab_hbm, tab_v, st).wait()

        def item(k, carry):
            t = t0 + 4 * k

            @pl.when(t < nt)
            def _():
                def prefill(j, cp):
                    for r in range(8):
                        for c in range(0, m, 16):
                            v = tab_v[4 * t + r // 2,
                                      pl.ds((r % 2) * m + c, 16)]
                            obs[0][j, r, pl.ds(m + c, 16)] = v
                            obs[1][j, r, pl.ds(m + c, 16)] = v
                    return cp

                lax.fori_loop(0, 16, prefill, 0)
                in_cp(0, t, 0).wait()
                in_cp(1, t, 1).wait()

                def body(buf, g):
                    @pl.when(g >= 2)
                    def _():
                        for j in range(16):
                            out_cp(buf, t, b0 + (g - 2) * 16 + j, j).wait()
                    gbase = g * 16

                    def sweep(jj, cp):
                        for u in range(2):
                            perm_j = lax.rem(lane + (2 * jj + u), 16)
                            bidx = gbase + perm_j
                            for r2 in range(0, 8, 2):
                                vs = []
                                for r in (r2, r2 + 1):
                                    for h in range(2):
                                        for c in range(0, m // 2, 16):
                                            cc = h * (m // 2) + c
                                            vs.append((r, cc, plsc.load_gather(
                                                xbs[h].at[r],
                                                [c + lane, bidx])))
                                for r, cc, v in vs:
                                    plsc.store_scatter(
                                        obs[buf],
                                        [perm_j,
                                         jnp.full((16,), r, jnp.int32),
                                         cc + lane], v)
                        return cp

                    lax.fori_loop(0, 8, sweep, 0)
                    for j in range(16):
                        out_cp(buf, t, b0 + gbase + j, j).start()

                def group(g, carry2):
                    @pl.when(lax.rem(g, 2) == 0)
                    def _():
                        body(0, g)

                    @pl.when(lax.rem(g, 2) == 1)
                    def _():
                        body(1, g)

                    return carry2

                lax.fori_loop(0, 8, group, 0)

                @pl.when(t + 4 < nt)
                def _():
                    in_cp(0, t + 4, 0).start()
                    in_cp(1, t + 4, 1).start()

                for j in range(16):
                    out_cp(0, t, b0 + 6 * 16 + j, j).wait()
                    out_cp(1, t, b0 + 7 * 16 + j, j).wait()

            return carry

        lax.fori_loop(0, 7, item, 0)

    return run(xt, tab2)
